# baseline (speedup 1.0000x reference)
import jax
import jax.numpy as jnp
from jax.experimental import pallas as pl
from jax.experimental.pallas import tpu as pltpu

_NUM_CLASSES = 80
_IGNORE_THRESH = 0.5


def _body(z_ref, t_ref, anc_ref, out_ref):
    t = t_ref[...]
    B = t.shape[1]
    T = t.shape[2]
    G = z_ref.shape[2]
    A = 3

    tsum = t[0] + t[1] + t[2] + t[3] + t[4]
    valid = tsum != 0.0
    gx = t[1] * G
    gy = t[2] * G
    gw = t[3] * G
    gh = t[4] * G
    gi = gx.astype(jnp.int32)
    gj = gy.astype(jnp.int32)

    ious = []
    for a in range(A):
        aw = anc_ref[a, 0]
        ah = anc_ref[a, 1]
        iw = jnp.clip(jnp.minimum(gw / 2, aw / 2) - jnp.maximum(-gw / 2, -aw / 2) + 1.0, 0.0, None)
        ih = jnp.clip(jnp.minimum(gh / 2, ah / 2) - jnp.maximum(-gh / 2, -ah / 2) + 1.0, 0.0, None)
        inter = iw * ih
        a1 = (gw + 1.0) * (gh + 1.0)
        a2 = (aw + 1.0) * (ah + 1.0)
        ious.append(inter / (a1 + a2 - inter + 1e-16))
    i0, i1, i2 = ious
    b01 = i1 > i0
    best_iou = jnp.where(b01, i1, i0)
    best_n = jnp.where(b01, 1, 0)
    b2 = i2 > best_iou
    best_iou = jnp.where(b2, i2, best_iou)
    best_n = jnp.where(b2, 2, best_n)
    cond_ign = valid & (best_iou > _IGNORE_THRESH)

    hi = jnp.where(cond_ign, best_n * G + gj, A * G)
    row_iota = jax.lax.broadcasted_iota(jnp.int32, (B, T, A * G), 2)
    u = jnp.where(hi == row_iota, 1.0, 0.0)
    col_iota = jax.lax.broadcasted_iota(jnp.int32, (B, T, G), 2)
    v = jnp.where(gi == col_iota, 1.0, 0.0)

    count = jax.lax.dot_general(
        u, v,
        dimension_numbers=(((1,), (1,)), ((0,), (0,))),
        preferred_element_type=jnp.float32,
    )

    z = z_ref[...]
    s = jax.nn.sigmoid(z)
    f = -jnp.maximum(jnp.log(1.0 - s), -100.0)
    out_ref[0, 0] = jnp.sum(jnp.where(count < 0.5, f, 0.0))


def kernel(pred, target, anchors, num_anchors, grid_size):
    B, C, G, _ = pred.shape
    A = anchors.shape[0]
    attrs = C // A
    stride = grid_size // G
    scaled_anchors = (anchors / stride) * (num_anchors // A)
    tgt = jnp.transpose(target, (2, 0, 1))[..., None]
    conf = pred.reshape(B, A, attrs, G, G)[:, :, 4]
    conf = conf.reshape(B, A * G, G)

    out = pl.pallas_call(
        _body,
        grid=(1,),
        out_shape=jax.ShapeDtypeStruct((1, 1), jnp.float32),
        in_specs=[
            pl.BlockSpec(conf.shape, lambda i: (0, 0, 0)),
            pl.BlockSpec(tgt.shape, lambda i: (0, 0, 0, 0)),
            pl.BlockSpec(memory_space=pltpu.SMEM),
        ],
        out_specs=pl.BlockSpec(memory_space=pltpu.SMEM),
    )(conf, tgt, scaled_anchors)
    return out[0, 0]


# speedup vs baseline: 3.6350x; 3.6350x over previous
import jax
import jax.numpy as jnp
from jax.experimental import pallas as pl
from jax.experimental.pallas import tpu as pltpu

_NUM_CLASSES = 80
_IGNORE_THRESH = 0.5


def _make_body(B, T, G, A, attrs):
    def _body(pred_ref, t_ref, anc_ref, out_ref, z_scr, sem):
        copies = []
        for b in range(B):
            for a in range(A):
                c = pltpu.make_async_copy(
                    pred_ref.at[b, a * attrs + 4], z_scr.at[b, a], sem)
                c.start()
                copies.append(c)

        t = t_ref[...]
        tsum = (t[:, :, 0:1] + t[:, :, 1:2] + t[:, :, 2:3]
                + t[:, :, 3:4] + t[:, :, 4:5])
        valid = tsum != 0.0
        gx = t[:, :, 1:2] * G
        gy = t[:, :, 2:3] * G
        gw = t[:, :, 3:4] * G
        gh = t[:, :, 4:5] * G
        gi = gx.astype(jnp.int32)
        gj = gy.astype(jnp.int32)

        ious = []
        for a in range(A):
            aw = anc_ref[a, 0]
            ah = anc_ref[a, 1]
            iw = jnp.clip(jnp.minimum(gw / 2, aw / 2) - jnp.maximum(-gw / 2, -aw / 2) + 1.0, 0.0, None)
            ih = jnp.clip(jnp.minimum(gh / 2, ah / 2) - jnp.maximum(-gh / 2, -ah / 2) + 1.0, 0.0, None)
            inter = iw * ih
            a1 = (gw + 1.0) * (gh + 1.0)
            a2 = (aw + 1.0) * (ah + 1.0)
            ious.append(inter / (a1 + a2 - inter + 1e-16))
        i0, i1, i2 = ious
        b01 = i1 > i0
        best_iou = jnp.where(b01, i1, i0)
        best_n = jnp.where(b01, 1, 0)
        b2 = i2 > best_iou
        best_iou = jnp.where(b2, i2, best_iou)
        best_n = jnp.where(b2, 2, best_n)
        cond_ign = valid & (best_iou > _IGNORE_THRESH)

        hi = jnp.where(cond_ign, best_n * G + gj, A * G)
        row_iota = jax.lax.broadcasted_iota(jnp.int32, (B, T, A * G), 2)
        u = jnp.where(hi == row_iota, 1.0, 0.0)
        col_iota = jax.lax.broadcasted_iota(jnp.int32, (B, T, G), 2)
        v = jnp.where(gi == col_iota, 1.0, 0.0)

        count = jax.lax.dot_general(
            u, v,
            dimension_numbers=(((1,), (1,)), ((0,), (0,))),
            preferred_element_type=jnp.float32,
        )

        for c in copies:
            c.wait()

        total = jnp.float32(0.0)
        for a in range(A):
            z = z_scr[:, a]
            s = jax.nn.sigmoid(z)
            f = -jnp.maximum(jnp.log(1.0 - s), -100.0)
            keep = count[:, a * G:(a + 1) * G, :] < 0.5
            total = total + jnp.sum(jnp.where(keep, f, 0.0))
        out_ref[0, 0] = total
    return _body


def kernel(pred, target, anchors, num_anchors, grid_size):
    B, C, G, _ = pred.shape
    A = anchors.shape[0]
    T = target.shape[1]
    attrs = C // A
    scaled_anchors = (anchors / (grid_size // G)) * (num_anchors // A)

    out = pl.pallas_call(
        _make_body(B, T, G, A, attrs),
        grid=(1,),
        out_shape=jax.ShapeDtypeStruct((1, 1), jnp.float32),
        in_specs=[
            pl.BlockSpec(memory_space=pl.ANY),
            pl.BlockSpec(target.shape, lambda i: (0, 0, 0)),
            pl.BlockSpec(memory_space=pltpu.SMEM),
        ],
        out_specs=pl.BlockSpec(memory_space=pltpu.SMEM),
        scratch_shapes=[
            pltpu.VMEM((B, A, G, G), jnp.float32),
            pltpu.SemaphoreType.DMA,
        ],
    )(pred, target, scaled_anchors)
    return out[0, 0]
